# Initial kernel scaffold; baseline (speedup 1.0000x reference)
import jax
import jax.numpy as jnp
from jax.experimental import pallas as pl


def kernel(x, edge_index, W1, b1, W2, b2):
    raise NotImplementedError("write your pallas kernel here")


# speedup vs baseline: 25.7031x; 25.7031x over previous
import functools

import jax
import jax.numpy as jnp
from jax import lax
from jax.experimental import pallas as pl
from jax.experimental.pallas import tpu as pltpu
from jax.experimental.pallas import tpu_sc as plsc

NC = 2
NS = 16
NW = NC * NS
LANES = 128
F = 16


def _mesh():
    return plsc.VectorSubcoreMesh(
        core_axis_name="c", subcore_axis_name="s", num_cores=NC, num_subcores=NS
    )


@functools.partial(jax.jit, static_argnames=("n_pad", "kpw"))
def _deg_pass(dst2d, n_pad, kpw):
    rpt = n_pad // NS

    def body(dst_hbm, out_hbm, dst_v, ones_v, stage_v, acc_sh):
        c = lax.axis_index("c")
        s = lax.axis_index("s")
        w = s * NC + c

        def zr(i, carry):
            stage_v[i] = jnp.zeros((F,), jnp.float32)
            return carry

        lax.fori_loop(0, rpt, zr, 0)

        def on(i, carry):
            ones_v[i] = jnp.ones((F,), jnp.float32)
            return carry

        lax.fori_loop(0, LANES, on, 0)
        pltpu.sync_copy(stage_v, acc_sh.at[pl.ds(s * rpt, rpt)])
        plsc.subcore_barrier()

        pltpu.sync_copy(dst_hbm.at[pl.ds(w * kpw, kpw)], dst_v)

        def step(j, carry):
            pltpu.sync_copy(ones_v, acc_sh.at[dst_v.at[j]], add=True)
            return carry

        lax.fori_loop(0, kpw, step, 0)
        plsc.subcore_barrier()
        pltpu.sync_copy(acc_sh.at[pl.ds(s * rpt, rpt)], stage_v)
        pltpu.sync_copy(stage_v, out_hbm.at[c, pl.ds(s * rpt, rpt)])

    return pl.kernel(
        body,
        out_type=jax.ShapeDtypeStruct((NC, n_pad, F), jnp.float32),
        mesh=_mesh(),
        compiler_params=pltpu.CompilerParams(use_tc_tiling_on_sc=False),
        scratch_types=[
            pltpu.VMEM((kpw, LANES), jnp.int32),
            pltpu.VMEM((LANES, F), jnp.float32),
            pltpu.VMEM((rpt, F), jnp.float32),
            pltpu.VMEM_SHARED((n_pad, F), jnp.float32),
        ],
    )(dst2d)


@functools.partial(jax.jit, static_argnames=("n_pad", "kpw"))
def _agg_pass(hs, src2d, dst2d, n_pad, kpw):
    rpt = n_pad // NS

    def body(hs_hbm, src_hbm, dst_hbm, out_hbm, src_v, dst_v, rows_v, stage_v,
             acc_sh, sem):
        c = lax.axis_index("c")
        s = lax.axis_index("s")
        w = s * NC + c

        def zr(i, carry):
            stage_v[i] = jnp.zeros((F,), jnp.float32)
            return carry

        lax.fori_loop(0, rpt, zr, 0)
        pltpu.sync_copy(stage_v, acc_sh.at[pl.ds(s * rpt, rpt)])
        plsc.subcore_barrier()

        pltpu.sync_copy(src_hbm.at[pl.ds(w * kpw, kpw)], src_v)
        pltpu.sync_copy(dst_hbm.at[pl.ds(w * kpw, kpw)], dst_v)

        def step(j, carry):
            pltpu.async_copy(hs_hbm.at[src_v.at[j]], rows_v, sem).wait()
            pltpu.sync_copy(rows_v, acc_sh.at[dst_v.at[j]], add=True)
            return carry

        lax.fori_loop(0, kpw, step, 0)
        plsc.subcore_barrier()
        pltpu.sync_copy(acc_sh.at[pl.ds(s * rpt, rpt)], stage_v)
        pltpu.sync_copy(stage_v, out_hbm.at[c, pl.ds(s * rpt, rpt)])

    return pl.kernel(
        body,
        out_type=jax.ShapeDtypeStruct((NC, n_pad, F), jnp.float32),
        mesh=_mesh(),
        compiler_params=pltpu.CompilerParams(use_tc_tiling_on_sc=False),
        scratch_types=[
            pltpu.VMEM((kpw, LANES), jnp.int32),
            pltpu.VMEM((kpw, LANES), jnp.int32),
            pltpu.VMEM((LANES, F), jnp.float32),
            pltpu.VMEM((rpt, F), jnp.float32),
            pltpu.VMEM_SHARED((n_pad, F), jnp.float32),
            pltpu.SemaphoreType.DMA,
        ],
    )(hs, src2d, dst2d)


RB = 640


def _tc1_body(x_ref, w_ref, d0_ref, d1_ref, hs_ref, self_ref, dis_ref):
    deg = d0_ref[...] + d1_ref[...] + 1.0
    dis = lax.rsqrt(deg)
    h = jnp.dot(x_ref[...], w_ref[...], preferred_element_type=jnp.float32)
    hs_ref[...] = h * dis
    self_ref[...] = h * dis * dis
    dis_ref[...] = dis


def _tc2_body(t0_ref, t1_ref, dis_ref, self_ref, b_ref, hs_ref, self2_ref):
    dis = dis_ref[...]
    s1 = dis * (t0_ref[...] + t1_ref[...]) + self_ref[...] + b_ref[...]
    r = jnp.maximum(s1, 0.0)
    hs_ref[...] = dis * r
    self2_ref[...] = dis * dis * r


def _tc3_body(t0_ref, t1_ref, dis_ref, self_ref, w_ref, b_ref, out_ref):
    s2 = dis_ref[...] * (t0_ref[...] + t1_ref[...]) + self_ref[...]
    logits = (
        jnp.dot(s2, w_ref[...], preferred_element_type=jnp.float32) + b_ref[...]
    )
    m = jnp.max(logits, axis=1, keepdims=True)
    e = jnp.exp(logits - m)
    lse = jnp.log(jnp.sum(e, axis=1, keepdims=True))
    out_ref[...] = (logits - m) - lse


def _row_spec(cols):
    return pl.BlockSpec((RB, cols), lambda i: (i, 0))


def _full_spec(shape):
    return pl.BlockSpec(shape, lambda i: tuple(0 for _ in shape))


def kernel(x, edge_index, W1, b1, W2, b2):
    n, d = x.shape
    h_dim = W1.shape[1]
    c_dim = W2.shape[1]
    e = edge_index.shape[1]
    assert h_dim == F

    n_pad = ((n + 1 + RB - 1) // RB) * RB
    kpw = ((-(-e // (NW * LANES)) + 7) // 8) * 8
    e_pad = NW * kpw * LANES

    src = jnp.pad(edge_index[0], (0, e_pad - e), constant_values=n)
    dst = jnp.pad(edge_index[1], (0, e_pad - e), constant_values=n)
    src2d = src.reshape(NW * kpw, LANES)
    dst2d = dst.reshape(NW * kpw, LANES)
    x_p = jnp.pad(x, ((0, n_pad - n), (0, 0)))

    grid = (n_pad // RB,)

    degp = _deg_pass(dst2d, n_pad=n_pad, kpw=kpw)

    hs1, self1, dis = pl.pallas_call(
        _tc1_body,
        grid=grid,
        in_specs=[
            _row_spec(d),
            _full_spec((d, F)),
            _row_spec(F),
            _row_spec(F),
        ],
        out_specs=[_row_spec(F)] * 3,
        out_shape=[jax.ShapeDtypeStruct((n_pad, F), jnp.float32)] * 3,
    )(x_p, W1, degp[0], degp[1])

    t1 = _agg_pass(hs1, src2d, dst2d, n_pad=n_pad, kpw=kpw)

    hs2, self2 = pl.pallas_call(
        _tc2_body,
        grid=grid,
        in_specs=[
            _row_spec(F),
            _row_spec(F),
            _row_spec(F),
            _row_spec(F),
            _full_spec((1, F)),
        ],
        out_specs=[_row_spec(F)] * 2,
        out_shape=[jax.ShapeDtypeStruct((n_pad, F), jnp.float32)] * 2,
    )(t1[0], t1[1], dis, self1, b1.reshape(1, F))

    t2 = _agg_pass(hs2, src2d, dst2d, n_pad=n_pad, kpw=kpw)

    out = pl.pallas_call(
        _tc3_body,
        grid=grid,
        in_specs=[
            _row_spec(F),
            _row_spec(F),
            _row_spec(F),
            _row_spec(F),
            _full_spec((F, c_dim)),
            _full_spec((1, c_dim)),
        ],
        out_specs=_row_spec(c_dim),
        out_shape=jax.ShapeDtypeStruct((n_pad, c_dim), jnp.float32),
    )(t2[0], t2[1], dis, self2, W2, b2.reshape(1, c_dim))

    return out[:n]


# speedup vs baseline: 29.7483x; 1.1574x over previous
import functools

import jax
import jax.numpy as jnp
from jax import lax
from jax.experimental import pallas as pl
from jax.experimental.pallas import tpu as pltpu
from jax.experimental.pallas import tpu_sc as plsc

NC = 2
NS = 16
NW = NC * NS
LANES = 128
F = 16
NBUF = 4


def _mesh():
    return plsc.VectorSubcoreMesh(
        core_axis_name="c", subcore_axis_name="s", num_cores=NC, num_subcores=NS
    )


@functools.partial(jax.jit, static_argnames=("n_pad", "kpw"))
def _deg_pass(dst2d, n_pad, kpw):
    rpt = n_pad // NS

    def body(dst_hbm, out_hbm, dst_v, ones_v, stage_v, acc_sh, sem):
        c = lax.axis_index("c")
        s = lax.axis_index("s")
        w = s * NC + c

        def zr(i, carry):
            stage_v[i] = jnp.zeros((F,), jnp.float32)
            return carry

        lax.fori_loop(0, rpt, zr, 0)

        def on(i, carry):
            ones_v[i] = jnp.ones((F,), jnp.float32)
            return carry

        lax.fori_loop(0, LANES, on, 0)
        pltpu.sync_copy(stage_v, acc_sh.at[pl.ds(s * rpt, rpt)])
        plsc.subcore_barrier()

        pltpu.sync_copy(dst_hbm.at[pl.ds(w * kpw, kpw)], dst_v)

        def step(t, carry):
            j0 = t * NBUF
            descs = [
                pltpu.async_copy(
                    ones_v, acc_sh.at[dst_v.at[j0 + b]], sem, add=True
                )
                for b in range(NBUF)
            ]
            for desc in descs:
                desc.wait()
            return carry

        lax.fori_loop(0, kpw // NBUF, step, 0)
        plsc.subcore_barrier()
        pltpu.sync_copy(acc_sh.at[pl.ds(s * rpt, rpt)], stage_v)
        pltpu.sync_copy(stage_v, out_hbm.at[c, pl.ds(s * rpt, rpt)])

    return pl.kernel(
        body,
        out_type=jax.ShapeDtypeStruct((NC, n_pad, F), jnp.float32),
        mesh=_mesh(),
        compiler_params=pltpu.CompilerParams(use_tc_tiling_on_sc=False),
        scratch_types=[
            pltpu.VMEM((kpw, LANES), jnp.int32),
            pltpu.VMEM((LANES, F), jnp.float32),
            pltpu.VMEM((rpt, F), jnp.float32),
            pltpu.VMEM_SHARED((n_pad, F), jnp.float32),
            pltpu.SemaphoreType.DMA,
        ],
    )(dst2d)


@functools.partial(jax.jit, static_argnames=("n_pad", "kpw"))
def _agg_pass(hs, src2d, dst2d, n_pad, kpw):
    rpt = n_pad // NS

    def body(hs_hbm, src_hbm, dst_hbm, out_hbm, src_v, dst_v, rows_v, stage_v,
             acc_sh, gsem, ssem):
        c = lax.axis_index("c")
        s = lax.axis_index("s")
        w = s * NC + c

        def zr(i, carry):
            stage_v[i] = jnp.zeros((F,), jnp.float32)
            return carry

        lax.fori_loop(0, rpt, zr, 0)
        pltpu.sync_copy(stage_v, acc_sh.at[pl.ds(s * rpt, rpt)])
        plsc.subcore_barrier()

        pltpu.sync_copy(src_hbm.at[pl.ds(w * kpw, kpw)], src_v)
        pltpu.sync_copy(dst_hbm.at[pl.ds(w * kpw, kpw)], dst_v)

        def step(t, carry):
            j0 = t * NBUF
            gds = [
                pltpu.async_copy(
                    hs_hbm.at[src_v.at[j0 + b]],
                    rows_v.at[pl.ds(b * LANES, LANES)],
                    gsem,
                )
                for b in range(NBUF)
            ]
            for desc in gds:
                desc.wait()
            sds = [
                pltpu.async_copy(
                    rows_v.at[pl.ds(b * LANES, LANES)],
                    acc_sh.at[dst_v.at[j0 + b]],
                    ssem,
                    add=True,
                )
                for b in range(NBUF)
            ]
            for desc in sds:
                desc.wait()
            return carry

        lax.fori_loop(0, kpw // NBUF, step, 0)
        plsc.subcore_barrier()
        pltpu.sync_copy(acc_sh.at[pl.ds(s * rpt, rpt)], stage_v)
        pltpu.sync_copy(stage_v, out_hbm.at[c, pl.ds(s * rpt, rpt)])

    return pl.kernel(
        body,
        out_type=jax.ShapeDtypeStruct((NC, n_pad, F), jnp.float32),
        mesh=_mesh(),
        compiler_params=pltpu.CompilerParams(use_tc_tiling_on_sc=False),
        scratch_types=[
            pltpu.VMEM((kpw, LANES), jnp.int32),
            pltpu.VMEM((kpw, LANES), jnp.int32),
            pltpu.VMEM((NBUF * LANES, F), jnp.float32),
            pltpu.VMEM((rpt, F), jnp.float32),
            pltpu.VMEM_SHARED((n_pad, F), jnp.float32),
            pltpu.SemaphoreType.DMA,
            pltpu.SemaphoreType.DMA,
        ],
    )(hs, src2d, dst2d)


RB = 640


def _tc1_body(x_ref, w_ref, d0_ref, d1_ref, hs_ref, self_ref, dis_ref):
    deg = d0_ref[...] + d1_ref[...] + 1.0
    dis = lax.rsqrt(deg)
    h = jnp.dot(x_ref[...], w_ref[...], preferred_element_type=jnp.float32)
    hs_ref[...] = h * dis
    self_ref[...] = h * dis * dis
    dis_ref[...] = dis


def _tc2_body(t0_ref, t1_ref, dis_ref, self_ref, b_ref, hs_ref, self2_ref):
    dis = dis_ref[...]
    s1 = dis * (t0_ref[...] + t1_ref[...]) + self_ref[...] + b_ref[...]
    r = jnp.maximum(s1, 0.0)
    hs_ref[...] = dis * r
    self2_ref[...] = dis * dis * r


def _tc3_body(t0_ref, t1_ref, dis_ref, self_ref, w_ref, b_ref, out_ref):
    s2 = dis_ref[...] * (t0_ref[...] + t1_ref[...]) + self_ref[...]
    logits = (
        jnp.dot(s2, w_ref[...], preferred_element_type=jnp.float32) + b_ref[...]
    )
    m = jnp.max(logits, axis=1, keepdims=True)
    e = jnp.exp(logits - m)
    lse = jnp.log(jnp.sum(e, axis=1, keepdims=True))
    out_ref[...] = (logits - m) - lse


def _row_spec(cols):
    return pl.BlockSpec((RB, cols), lambda i: (i, 0))


def _full_spec(shape):
    return pl.BlockSpec(shape, lambda i: tuple(0 for _ in shape))


def kernel(x, edge_index, W1, b1, W2, b2):
    n, d = x.shape
    h_dim = W1.shape[1]
    c_dim = W2.shape[1]
    e = edge_index.shape[1]
    assert h_dim == F

    n_pad = ((n + 1 + RB - 1) // RB) * RB
    kpw = ((-(-e // (NW * LANES)) + 7) // 8) * 8
    e_pad = NW * kpw * LANES

    src = jnp.pad(edge_index[0], (0, e_pad - e), constant_values=n)
    dst = jnp.pad(edge_index[1], (0, e_pad - e), constant_values=n)
    src2d = src.reshape(NW * kpw, LANES)
    dst2d = dst.reshape(NW * kpw, LANES)
    x_p = jnp.pad(x, ((0, n_pad - n), (0, 0)))

    grid = (n_pad // RB,)

    degp = _deg_pass(dst2d, n_pad=n_pad, kpw=kpw)

    hs1, self1, dis = pl.pallas_call(
        _tc1_body,
        grid=grid,
        in_specs=[
            _row_spec(d),
            _full_spec((d, F)),
            _row_spec(F),
            _row_spec(F),
        ],
        out_specs=[_row_spec(F)] * 3,
        out_shape=[jax.ShapeDtypeStruct((n_pad, F), jnp.float32)] * 3,
    )(x_p, W1, degp[0], degp[1])

    t1 = _agg_pass(hs1, src2d, dst2d, n_pad=n_pad, kpw=kpw)

    hs2, self2 = pl.pallas_call(
        _tc2_body,
        grid=grid,
        in_specs=[
            _row_spec(F),
            _row_spec(F),
            _row_spec(F),
            _row_spec(F),
            _full_spec((1, F)),
        ],
        out_specs=[_row_spec(F)] * 2,
        out_shape=[jax.ShapeDtypeStruct((n_pad, F), jnp.float32)] * 2,
    )(t1[0], t1[1], dis, self1, b1.reshape(1, F))

    t2 = _agg_pass(hs2, src2d, dst2d, n_pad=n_pad, kpw=kpw)

    out = pl.pallas_call(
        _tc3_body,
        grid=grid,
        in_specs=[
            _row_spec(F),
            _row_spec(F),
            _row_spec(F),
            _row_spec(F),
            _full_spec((F, c_dim)),
            _full_spec((1, c_dim)),
        ],
        out_specs=_row_spec(c_dim),
        out_shape=jax.ShapeDtypeStruct((n_pad, c_dim), jnp.float32),
    )(t2[0], t2[1], dis, self2, W2, b2.reshape(1, c_dim))

    return out[:n]


# speedup vs baseline: 30.5515x; 1.0270x over previous
import functools

import jax
import jax.numpy as jnp
from jax import lax
from jax.experimental import pallas as pl
from jax.experimental.pallas import tpu as pltpu
from jax.experimental.pallas import tpu_sc as plsc

NC = 2
NS = 16
NW = NC * NS
LANES = 128
F = 16
NBUF = 8


def _mesh():
    return plsc.VectorSubcoreMesh(
        core_axis_name="c", subcore_axis_name="s", num_cores=NC, num_subcores=NS
    )


@functools.partial(jax.jit, static_argnames=("n_pad", "kpw"))
def _deg_pass(dst2d, n_pad, kpw):
    rpt = n_pad // NS

    def body(dst_hbm, out_hbm, dst_v, ones_v, stage_v, acc_sh, sem):
        c = lax.axis_index("c")
        s = lax.axis_index("s")
        w = s * NC + c

        def zr(i, carry):
            stage_v[i] = jnp.zeros((F,), jnp.float32)
            return carry

        lax.fori_loop(0, rpt, zr, 0)

        def on(i, carry):
            ones_v[i] = jnp.ones((F,), jnp.float32)
            return carry

        lax.fori_loop(0, LANES, on, 0)
        pltpu.sync_copy(stage_v, acc_sh.at[pl.ds(s * rpt, rpt)])
        plsc.subcore_barrier()

        pltpu.sync_copy(dst_hbm.at[pl.ds(w * kpw, kpw)], dst_v)

        def step(t, carry):
            j0 = t * NBUF
            descs = [
                pltpu.async_copy(
                    ones_v, acc_sh.at[dst_v.at[j0 + b]], sem, add=True
                )
                for b in range(NBUF)
            ]
            for desc in descs:
                desc.wait()
            return carry

        lax.fori_loop(0, kpw // NBUF, step, 0)
        plsc.subcore_barrier()
        pltpu.sync_copy(acc_sh.at[pl.ds(s * rpt, rpt)], stage_v)
        pltpu.sync_copy(stage_v, out_hbm.at[c, pl.ds(s * rpt, rpt)])

    return pl.kernel(
        body,
        out_type=jax.ShapeDtypeStruct((NC, n_pad, F), jnp.float32),
        mesh=_mesh(),
        compiler_params=pltpu.CompilerParams(use_tc_tiling_on_sc=False),
        scratch_types=[
            pltpu.VMEM((kpw, LANES), jnp.int32),
            pltpu.VMEM((LANES, F), jnp.float32),
            pltpu.VMEM((rpt, F), jnp.float32),
            pltpu.VMEM_SHARED((n_pad, F), jnp.float32),
            pltpu.SemaphoreType.DMA,
        ],
    )(dst2d)


@functools.partial(jax.jit, static_argnames=("n_pad", "kpw"))
def _agg_pass(hs, src2d, dst2d, n_pad, kpw):
    rpt = n_pad // NS

    def body(hs_hbm, src_hbm, dst_hbm, out_hbm, src_v, dst_v, rows_v, stage_v,
             acc_sh, gsem, ssem):
        c = lax.axis_index("c")
        s = lax.axis_index("s")
        w = s * NC + c

        def zr(i, carry):
            stage_v[i] = jnp.zeros((F,), jnp.float32)
            return carry

        lax.fori_loop(0, rpt, zr, 0)
        pltpu.sync_copy(stage_v, acc_sh.at[pl.ds(s * rpt, rpt)])
        plsc.subcore_barrier()

        pltpu.sync_copy(src_hbm.at[pl.ds(w * kpw, kpw)], src_v)
        pltpu.sync_copy(dst_hbm.at[pl.ds(w * kpw, kpw)], dst_v)

        def step(t, carry):
            j0 = t * NBUF
            gds = [
                pltpu.async_copy(
                    hs_hbm.at[src_v.at[j0 + b]],
                    rows_v.at[pl.ds(b * LANES, LANES)],
                    gsem,
                )
                for b in range(NBUF)
            ]
            for desc in gds:
                desc.wait()
            sds = [
                pltpu.async_copy(
                    rows_v.at[pl.ds(b * LANES, LANES)],
                    acc_sh.at[dst_v.at[j0 + b]],
                    ssem,
                    add=True,
                )
                for b in range(NBUF)
            ]
            for desc in sds:
                desc.wait()
            return carry

        lax.fori_loop(0, kpw // NBUF, step, 0)
        plsc.subcore_barrier()
        pltpu.sync_copy(acc_sh.at[pl.ds(s * rpt, rpt)], stage_v)
        pltpu.sync_copy(stage_v, out_hbm.at[c, pl.ds(s * rpt, rpt)])

    return pl.kernel(
        body,
        out_type=jax.ShapeDtypeStruct((NC, n_pad, F), jnp.float32),
        mesh=_mesh(),
        compiler_params=pltpu.CompilerParams(use_tc_tiling_on_sc=False),
        scratch_types=[
            pltpu.VMEM((kpw, LANES), jnp.int32),
            pltpu.VMEM((kpw, LANES), jnp.int32),
            pltpu.VMEM((NBUF * LANES, F), jnp.float32),
            pltpu.VMEM((rpt, F), jnp.float32),
            pltpu.VMEM_SHARED((n_pad, F), jnp.float32),
            pltpu.SemaphoreType.DMA,
            pltpu.SemaphoreType.DMA,
        ],
    )(hs, src2d, dst2d)


RB = 640


def _tc1_body(x_ref, w_ref, d0_ref, d1_ref, hs_ref, self_ref, dis_ref):
    deg = d0_ref[...] + d1_ref[...] + 1.0
    dis = lax.rsqrt(deg)
    h = jnp.dot(x_ref[...], w_ref[...], preferred_element_type=jnp.float32)
    hs_ref[...] = h * dis
    self_ref[...] = h * dis * dis
    dis_ref[...] = dis


def _tc2_body(t0_ref, t1_ref, dis_ref, self_ref, b_ref, hs_ref, self2_ref):
    dis = dis_ref[...]
    s1 = dis * (t0_ref[...] + t1_ref[...]) + self_ref[...] + b_ref[...]
    r = jnp.maximum(s1, 0.0)
    hs_ref[...] = dis * r
    self2_ref[...] = dis * dis * r


def _tc3_body(t0_ref, t1_ref, dis_ref, self_ref, w_ref, b_ref, out_ref):
    s2 = dis_ref[...] * (t0_ref[...] + t1_ref[...]) + self_ref[...]
    logits = (
        jnp.dot(s2, w_ref[...], preferred_element_type=jnp.float32) + b_ref[...]
    )
    m = jnp.max(logits, axis=1, keepdims=True)
    e = jnp.exp(logits - m)
    lse = jnp.log(jnp.sum(e, axis=1, keepdims=True))
    out_ref[...] = (logits - m) - lse


def _row_spec(cols):
    return pl.BlockSpec((RB, cols), lambda i: (i, 0))


def _full_spec(shape):
    return pl.BlockSpec(shape, lambda i: tuple(0 for _ in shape))


def kernel(x, edge_index, W1, b1, W2, b2):
    n, d = x.shape
    h_dim = W1.shape[1]
    c_dim = W2.shape[1]
    e = edge_index.shape[1]
    assert h_dim == F

    n_pad = ((n + 1 + RB - 1) // RB) * RB
    kpw = ((-(-e // (NW * LANES)) + 7) // 8) * 8
    e_pad = NW * kpw * LANES

    src = jnp.pad(edge_index[0], (0, e_pad - e), constant_values=n)
    dst = jnp.pad(edge_index[1], (0, e_pad - e), constant_values=n)
    src2d = src.reshape(NW * kpw, LANES)
    dst2d = dst.reshape(NW * kpw, LANES)
    x_p = jnp.pad(x, ((0, n_pad - n), (0, 0)))

    grid = (n_pad // RB,)

    degp = _deg_pass(dst2d, n_pad=n_pad, kpw=kpw)

    hs1, self1, dis = pl.pallas_call(
        _tc1_body,
        grid=grid,
        in_specs=[
            _row_spec(d),
            _full_spec((d, F)),
            _row_spec(F),
            _row_spec(F),
        ],
        out_specs=[_row_spec(F)] * 3,
        out_shape=[jax.ShapeDtypeStruct((n_pad, F), jnp.float32)] * 3,
    )(x_p, W1, degp[0], degp[1])

    t1 = _agg_pass(hs1, src2d, dst2d, n_pad=n_pad, kpw=kpw)

    hs2, self2 = pl.pallas_call(
        _tc2_body,
        grid=grid,
        in_specs=[
            _row_spec(F),
            _row_spec(F),
            _row_spec(F),
            _row_spec(F),
            _full_spec((1, F)),
        ],
        out_specs=[_row_spec(F)] * 2,
        out_shape=[jax.ShapeDtypeStruct((n_pad, F), jnp.float32)] * 2,
    )(t1[0], t1[1], dis, self1, b1.reshape(1, F))

    t2 = _agg_pass(hs2, src2d, dst2d, n_pad=n_pad, kpw=kpw)

    out = pl.pallas_call(
        _tc3_body,
        grid=grid,
        in_specs=[
            _row_spec(F),
            _row_spec(F),
            _row_spec(F),
            _row_spec(F),
            _full_spec((F, c_dim)),
            _full_spec((1, c_dim)),
        ],
        out_specs=_row_spec(c_dim),
        out_shape=jax.ShapeDtypeStruct((n_pad, c_dim), jnp.float32),
    )(t2[0], t2[1], dis, self2, W2, b2.reshape(1, c_dim))

    return out[:n]


# speedup vs baseline: 36.5017x; 1.1948x over previous
import functools

import jax
import jax.numpy as jnp
from jax import lax
from jax.experimental import pallas as pl
from jax.experimental.pallas import tpu as pltpu
from jax.experimental.pallas import tpu_sc as plsc

NC = 2
NS = 16
LANES = 128
F = 16
NBUF = 8
SPLIT0 = 0.65


def _mesh():
    return plsc.VectorSubcoreMesh(
        core_axis_name="c", subcore_axis_name="s", num_cores=NC, num_subcores=NS
    )


def _split(kpw_tot):
    kpw0 = min(
        kpw_tot - 8, max(8, int(round(kpw_tot * SPLIT0 / NBUF)) * NBUF)
    )
    return kpw0, kpw_tot - kpw0


@functools.partial(jax.jit, static_argnames=("n_pad", "kpw_tot"))
def _deg_pass(dst2d, n_pad, kpw_tot):
    rpt = n_pad // NS
    kpw0, kpw1 = _split(kpw_tot)

    def body(dst_hbm, out_hbm, dst_v, ones_v, stage_v, acc_sh, sem):
        c = lax.axis_index("c")
        s = lax.axis_index("s")

        def zr(i, carry):
            stage_v[i] = jnp.zeros((F,), jnp.float32)
            return carry

        lax.fori_loop(0, rpt, zr, 0)

        def on(i, carry):
            ones_v[i] = jnp.ones((F,), jnp.float32)
            return carry

        lax.fori_loop(0, LANES, on, 0)
        pltpu.sync_copy(stage_v, acc_sh.at[pl.ds(s * rpt, rpt)])
        plsc.subcore_barrier()

        def run(base, kpw):
            pltpu.sync_copy(
                dst_hbm.at[pl.ds(base, kpw)], dst_v.at[pl.ds(0, kpw)]
            )

            def step(t, carry):
                j0 = t * NBUF
                descs = [
                    pltpu.async_copy(
                        ones_v, acc_sh.at[dst_v.at[j0 + b]], sem, add=True
                    )
                    for b in range(NBUF)
                ]
                for desc in descs:
                    desc.wait()
                return carry

            lax.fori_loop(0, kpw // NBUF, step, 0)

        @pl.when(c == 0)
        def _():
            run(s * kpw0, kpw0)

        @pl.when(c == 1)
        def _():
            run(NS * kpw0 + s * kpw1, kpw1)

        plsc.subcore_barrier()
        pltpu.sync_copy(acc_sh.at[pl.ds(s * rpt, rpt)], stage_v)
        pltpu.sync_copy(stage_v, out_hbm.at[c, pl.ds(s * rpt, rpt)])

    return pl.kernel(
        body,
        out_type=jax.ShapeDtypeStruct((NC, n_pad, F), jnp.float32),
        mesh=_mesh(),
        compiler_params=pltpu.CompilerParams(use_tc_tiling_on_sc=False),
        scratch_types=[
            pltpu.VMEM((max(kpw0, kpw1), LANES), jnp.int32),
            pltpu.VMEM((LANES, F), jnp.float32),
            pltpu.VMEM((rpt, F), jnp.float32),
            pltpu.VMEM_SHARED((n_pad, F), jnp.float32),
            pltpu.SemaphoreType.DMA,
        ],
    )(dst2d)


@functools.partial(jax.jit, static_argnames=("n_pad", "kpw_tot"))
def _agg_pass(hs, src2d, dst2d, n_pad, kpw_tot):
    rpt = n_pad // NS
    kpw0, kpw1 = _split(kpw_tot)

    def body(hs_hbm, src_hbm, dst_hbm, out_hbm, src_v, dst_v, rows_v, stage_v,
             acc_sh, gsem, ssem):
        c = lax.axis_index("c")
        s = lax.axis_index("s")

        def zr(i, carry):
            stage_v[i] = jnp.zeros((F,), jnp.float32)
            return carry

        lax.fori_loop(0, rpt, zr, 0)
        pltpu.sync_copy(stage_v, acc_sh.at[pl.ds(s * rpt, rpt)])
        plsc.subcore_barrier()

        def run(base, kpw):
            pltpu.sync_copy(
                src_hbm.at[pl.ds(base, kpw)], src_v.at[pl.ds(0, kpw)]
            )
            pltpu.sync_copy(
                dst_hbm.at[pl.ds(base, kpw)], dst_v.at[pl.ds(0, kpw)]
            )

            def step(t, carry):
                j0 = t * NBUF
                gds = [
                    pltpu.async_copy(
                        hs_hbm.at[src_v.at[j0 + b]],
                        rows_v.at[pl.ds(b * LANES, LANES)],
                        gsem,
                    )
                    for b in range(NBUF)
                ]
                for desc in gds:
                    desc.wait()
                sds = [
                    pltpu.async_copy(
                        rows_v.at[pl.ds(b * LANES, LANES)],
                        acc_sh.at[dst_v.at[j0 + b]],
                        ssem,
                        add=True,
                    )
                    for b in range(NBUF)
                ]
                for desc in sds:
                    desc.wait()
                return carry

            lax.fori_loop(0, kpw // NBUF, step, 0)

        @pl.when(c == 0)
        def _():
            run(s * kpw0, kpw0)

        @pl.when(c == 1)
        def _():
            run(NS * kpw0 + s * kpw1, kpw1)

        plsc.subcore_barrier()
        pltpu.sync_copy(acc_sh.at[pl.ds(s * rpt, rpt)], stage_v)
        pltpu.sync_copy(stage_v, out_hbm.at[c, pl.ds(s * rpt, rpt)])

    return pl.kernel(
        body,
        out_type=jax.ShapeDtypeStruct((NC, n_pad, F), jnp.float32),
        mesh=_mesh(),
        compiler_params=pltpu.CompilerParams(use_tc_tiling_on_sc=False),
        scratch_types=[
            pltpu.VMEM((max(kpw0, kpw1), LANES), jnp.int32),
            pltpu.VMEM((max(kpw0, kpw1), LANES), jnp.int32),
            pltpu.VMEM((NBUF * LANES, F), jnp.float32),
            pltpu.VMEM((rpt, F), jnp.float32),
            pltpu.VMEM_SHARED((n_pad, F), jnp.float32),
            pltpu.SemaphoreType.DMA,
            pltpu.SemaphoreType.DMA,
        ],
    )(hs, src2d, dst2d)


RB = 640


def _mm1_body(x_ref, w_ref, h_ref):
    h_ref[...] = jnp.dot(
        x_ref[...], w_ref[...], preferred_element_type=jnp.float32
    )


def _mm2_body(s_ref, w_ref, b_ref, out_ref):
    logits = (
        jnp.dot(s_ref[...], w_ref[...], preferred_element_type=jnp.float32)
        + b_ref[...]
    )
    m = jnp.max(logits, axis=1, keepdims=True)
    e = jnp.exp(logits - m)
    lse = jnp.log(jnp.sum(e, axis=1, keepdims=True))
    out_ref[...] = (logits - m) - lse


def _row_spec(cols):
    return pl.BlockSpec((RB, cols), lambda i: (i, 0))


def _full_spec(shape):
    return pl.BlockSpec(shape, lambda i: tuple(0 for _ in shape))


def kernel(x, edge_index, W1, b1, W2, b2):
    n, d = x.shape
    c_dim = W2.shape[1]
    e = edge_index.shape[1]
    assert W1.shape[1] == F

    n_pad = ((n + 1 + RB - 1) // RB) * RB
    q = 2 * NBUF if (2 * NBUF) % 8 == 0 else 8 * NBUF
    kpw_tot = -(-e // (NS * LANES * q)) * q
    e_pad = NS * kpw_tot * LANES

    src = jnp.pad(edge_index[0], (0, e_pad - e), constant_values=n)
    dst = jnp.pad(edge_index[1], (0, e_pad - e), constant_values=n)
    src2d = src.reshape(NS * kpw_tot, LANES)
    dst2d = dst.reshape(NS * kpw_tot, LANES)
    x_p = jnp.pad(x, ((0, n_pad - n), (0, 0)))

    grid = (n_pad // RB,)

    degp = _deg_pass(dst2d, n_pad=n_pad, kpw_tot=kpw_tot)
    h1 = pl.pallas_call(
        _mm1_body,
        grid=grid,
        in_specs=[_row_spec(d), _full_spec((d, F))],
        out_specs=_row_spec(F),
        out_shape=jax.ShapeDtypeStruct((n_pad, F), jnp.float32),
    )(x_p, W1)

    dis = lax.rsqrt(degp[0] + degp[1] + 1.0)
    hs1 = dis * h1

    t1 = _agg_pass(hs1, src2d, dst2d, n_pad=n_pad, kpw_tot=kpw_tot)

    r = jnp.maximum(dis * (t1[0] + t1[1] + hs1) + b1, 0.0)
    hs2 = dis * r

    t2 = _agg_pass(hs2, src2d, dst2d, n_pad=n_pad, kpw_tot=kpw_tot)

    s2 = dis * (t2[0] + t2[1] + hs2)

    out = pl.pallas_call(
        _mm2_body,
        grid=grid,
        in_specs=[
            _row_spec(F),
            _full_spec((F, c_dim)),
            _full_spec((1, c_dim)),
        ],
        out_specs=_row_spec(c_dim),
        out_shape=jax.ShapeDtypeStruct((n_pad, c_dim), jnp.float32),
    )(s2, W2, b2.reshape(1, c_dim))

    return out[:n]


# speedup vs baseline: 47.2554x; 1.2946x over previous
import functools

import jax
import jax.numpy as jnp
from jax import lax
from jax.experimental import pallas as pl
from jax.experimental.pallas import tpu as pltpu
from jax.experimental.pallas import tpu_sc as plsc

NC = 2
NS = 16
LANES = 128
F = 16
NBUF = 4
SPLIT0 = 0.65


def _mesh():
    return plsc.VectorSubcoreMesh(
        core_axis_name="c", subcore_axis_name="s", num_cores=NC, num_subcores=NS
    )


def _partition(rows):
    kpw0 = max(NBUF, int(round(rows * SPLIT0 / (NS * NBUF))) * NBUF)
    kpw0 = min(kpw0, (rows // (NS * NBUF)) * NBUF)
    rem = rows - NS * kpw0
    kpw1 = max(0, (rem // NS) // NBUF * NBUF)
    extra = rows - NS * kpw0 - NS * kpw1
    return kpw0, kpw1, extra


def _edge_loop(run_batch, idx_stage, ei3, kpw0, kpw1, extra, c, s):
    def run(base, kpw):
        idx_stage(base, kpw, 0)

        def step(t, carry):
            run_batch(t * NBUF, NBUF)
            return carry

        lax.fori_loop(0, kpw // NBUF, step, 0)

    @pl.when(c == 0)
    def _():
        run(s * kpw0, kpw0)

    @pl.when(c == 1)
    def _():
        if kpw1 > 0:
            run(NS * kpw0 + s * kpw1, kpw1)

    if extra > 0:
        nfull = extra // NBUF
        tail = extra % NBUF

        @pl.when((c == 1) & (s == NS - 1))
        def _():
            idx_stage(NS * (kpw0 + kpw1), extra, 0)

            def step(t, carry):
                run_batch(t * NBUF, NBUF)
                return carry

            lax.fori_loop(0, nfull, step, 0)
            if tail:
                run_batch(nfull * NBUF, tail)


@functools.partial(jax.jit, static_argnames=("n_pad", "rows"))
def _deg_pass(ei3, n_pad, rows):
    rpt = n_pad // NS
    kpw0, kpw1, extra = _partition(rows)
    kmax = max(kpw0, kpw1, extra)

    def body(ei_hbm, out_hbm, dst_v, ones_v, stage_v, acc_sh, sem):
        c = lax.axis_index("c")
        s = lax.axis_index("s")

        def zr(i, carry):
            stage_v[i] = jnp.zeros((F,), jnp.float32)
            return carry

        lax.fori_loop(0, rpt, zr, 0)

        def on(i, carry):
            ones_v[i] = jnp.ones((F,), jnp.float32)
            return carry

        lax.fori_loop(0, LANES, on, 0)
        pltpu.sync_copy(stage_v, acc_sh.at[pl.ds(s * rpt, rpt)])
        plsc.subcore_barrier()

        def idx_stage(base, count, off):
            pltpu.sync_copy(
                ei_hbm.at[1, pl.ds(base, count)], dst_v.at[pl.ds(off, count)]
            )

        def run_batch(j0, nb):
            descs = [
                pltpu.async_copy(
                    ones_v, acc_sh.at[dst_v.at[j0 + b]], sem, add=True
                )
                for b in range(nb)
            ]
            for desc in descs:
                desc.wait()

        _edge_loop(run_batch, idx_stage, ei_hbm, kpw0, kpw1, extra, c, s)

        plsc.subcore_barrier()
        pltpu.sync_copy(acc_sh.at[pl.ds(s * rpt, rpt)], stage_v)
        pltpu.sync_copy(stage_v, out_hbm.at[c, pl.ds(s * rpt, rpt)])

    return pl.kernel(
        body,
        out_type=jax.ShapeDtypeStruct((NC, n_pad, F), jnp.float32),
        mesh=_mesh(),
        compiler_params=pltpu.CompilerParams(use_tc_tiling_on_sc=False),
        scratch_types=[
            pltpu.VMEM((kmax, LANES), jnp.int32),
            pltpu.VMEM((LANES, F), jnp.float32),
            pltpu.VMEM((rpt, F), jnp.float32),
            pltpu.VMEM_SHARED((n_pad, F), jnp.float32),
            pltpu.SemaphoreType.DMA,
        ],
    )(ei3)


@functools.partial(jax.jit, static_argnames=("n_pad", "rows"))
def _agg_pass(hs, ei3, n_pad, rows):
    rpt = n_pad // NS
    kpw0, kpw1, extra = _partition(rows)
    kmax = max(kpw0, kpw1, extra)

    def body(hs_hbm, ei_hbm, out_hbm, src_v, dst_v, rows_v, stage_v,
             acc_sh, gsem, ssem):
        c = lax.axis_index("c")
        s = lax.axis_index("s")

        def zr(i, carry):
            stage_v[i] = jnp.zeros((F,), jnp.float32)
            return carry

        lax.fori_loop(0, rpt, zr, 0)
        pltpu.sync_copy(stage_v, acc_sh.at[pl.ds(s * rpt, rpt)])
        plsc.subcore_barrier()

        def idx_stage(base, count, off):
            pltpu.sync_copy(
                ei_hbm.at[0, pl.ds(base, count)], src_v.at[pl.ds(off, count)]
            )
            pltpu.sync_copy(
                ei_hbm.at[1, pl.ds(base, count)], dst_v.at[pl.ds(off, count)]
            )

        def run_batch(j0, nb):
            gds = [
                pltpu.async_copy(
                    hs_hbm.at[src_v.at[j0 + b]],
                    rows_v.at[pl.ds(b * LANES, LANES)],
                    gsem,
                )
                for b in range(nb)
            ]
            for desc in gds:
                desc.wait()
            sds = [
                pltpu.async_copy(
                    rows_v.at[pl.ds(b * LANES, LANES)],
                    acc_sh.at[dst_v.at[j0 + b]],
                    ssem,
                    add=True,
                )
                for b in range(nb)
            ]
            for desc in sds:
                desc.wait()

        _edge_loop(run_batch, idx_stage, ei_hbm, kpw0, kpw1, extra, c, s)

        plsc.subcore_barrier()
        pltpu.sync_copy(acc_sh.at[pl.ds(s * rpt, rpt)], stage_v)
        pltpu.sync_copy(stage_v, out_hbm.at[c, pl.ds(s * rpt, rpt)])

    return pl.kernel(
        body,
        out_type=jax.ShapeDtypeStruct((NC, n_pad, F), jnp.float32),
        mesh=_mesh(),
        compiler_params=pltpu.CompilerParams(use_tc_tiling_on_sc=False),
        scratch_types=[
            pltpu.VMEM((kmax, LANES), jnp.int32),
            pltpu.VMEM((kmax, LANES), jnp.int32),
            pltpu.VMEM((NBUF * LANES, F), jnp.float32),
            pltpu.VMEM((rpt, F), jnp.float32),
            pltpu.VMEM_SHARED((n_pad, F), jnp.float32),
            pltpu.SemaphoreType.DMA,
            pltpu.SemaphoreType.DMA,
        ],
    )(hs, ei3)


def _mm1_body(x_ref, w_ref, h_ref):
    h_ref[...] = jnp.dot(
        x_ref[...], w_ref[...], preferred_element_type=jnp.float32
    )


def _mm2_body(s_ref, w_ref, b_ref, out_ref):
    logits = (
        jnp.dot(s_ref[...], w_ref[...], preferred_element_type=jnp.float32)
        + b_ref[...]
    )
    m = jnp.max(logits, axis=1, keepdims=True)
    e = jnp.exp(logits - m)
    lse = jnp.log(jnp.sum(e, axis=1, keepdims=True))
    out_ref[...] = (logits - m) - lse


def _full_spec(shape):
    return pl.BlockSpec(shape, lambda i: tuple(0 for _ in shape))


def kernel(x, edge_index, W1, b1, W2, b2):
    n, d = x.shape
    c_dim = W2.shape[1]
    e = edge_index.shape[1]
    assert W1.shape[1] == F

    if e % LANES == 0:
        ei = edge_index
        n_pad = -(-n // LANES) * LANES
    else:
        pad = LANES - e % LANES
        n_pad = (n + LANES) // LANES * LANES
        ei = jnp.concatenate(
            [
                edge_index,
                jnp.tile(
                    jnp.array([[0], [n]], jnp.int32), (1, pad)
                ),
            ],
            axis=1,
        )
    rows = ei.shape[1] // LANES
    ei3 = ei.reshape(2, rows, LANES)

    rb = next((b for b in (1000, 512, 256, 128, 8) if n % b == 0), None)
    x_mm = x
    n_mm = n
    if rb is None:
        rb = 512
        n_mm = -(-n // rb) * rb
        x_mm = jnp.pad(x, ((0, n_mm - n), (0, 0)))
    grid = (n_mm // rb,)

    def row_spec(cols):
        return pl.BlockSpec((rb, cols), lambda i: (i, 0))

    degp = _deg_pass(ei3, n_pad=n_pad, rows=rows)
    h1 = pl.pallas_call(
        _mm1_body,
        grid=grid,
        in_specs=[row_spec(d), _full_spec((d, F))],
        out_specs=row_spec(F),
        out_shape=jax.ShapeDtypeStruct((n_mm, F), jnp.float32),
    )(x_mm, W1)[:n]

    dis = lax.rsqrt(degp[0, :n] + degp[1, :n] + 1.0)
    hs1 = dis * h1

    t1 = _agg_pass(hs1, ei3, n_pad=n_pad, rows=rows)

    r = jnp.maximum(dis * (t1[0, :n] + t1[1, :n] + hs1) + b1, 0.0)
    hs2 = dis * r

    t2 = _agg_pass(hs2, ei3, n_pad=n_pad, rows=rows)

    s2 = dis * (t2[0, :n] + t2[1, :n] + hs2)
    if n_mm != n:
        s2 = jnp.pad(s2, ((0, n_mm - n), (0, 0)))

    out = pl.pallas_call(
        _mm2_body,
        grid=grid,
        in_specs=[
            row_spec(F),
            _full_spec((F, c_dim)),
            _full_spec((1, c_dim)),
        ],
        out_specs=row_spec(c_dim),
        out_shape=jax.ShapeDtypeStruct((n_mm, c_dim), jnp.float32),
    )(s2, W2, b2.reshape(1, c_dim))

    return out[:n]


# speedup vs baseline: 53.6800x; 1.1360x over previous
import functools

import jax
import jax.numpy as jnp
from jax import lax
from jax.experimental import pallas as pl
from jax.experimental.pallas import tpu as pltpu
from jax.experimental.pallas import tpu_sc as plsc

NC = 2
NS = 16
LANES = 128
F = 16
NBUF = 4
SPLIT0 = 0.54


def _mesh():
    return plsc.VectorSubcoreMesh(
        core_axis_name="c", subcore_axis_name="s", num_cores=NC, num_subcores=NS
    )


def _partition(rows):
    kpw0 = max(NBUF, int(round(rows * SPLIT0 / (NS * NBUF))) * NBUF)
    kpw0 = min(kpw0, (rows // (NS * NBUF)) * NBUF)
    rem = rows - NS * kpw0
    kpw1 = max(0, (rem // NS) // NBUF * NBUF)
    extra = rows - NS * kpw0 - NS * kpw1
    return kpw0, kpw1, extra


def _edge_loop(run_batch, idx_stage, ei3, kpw0, kpw1, extra, c, s):
    def run(base, kpw):
        idx_stage(base, kpw, 0)

        def step(t, carry):
            run_batch(t * NBUF, NBUF)
            return carry

        lax.fori_loop(0, kpw // NBUF, step, 0)

    @pl.when(c == 0)
    def _():
        run(s * kpw0, kpw0)

    @pl.when(c == 1)
    def _():
        if kpw1 > 0:
            run(NS * kpw0 + s * kpw1, kpw1)

    if extra > 0:
        nfull = extra // NBUF
        tail = extra % NBUF

        @pl.when((c == 1) & (s == NS - 1))
        def _():
            idx_stage(NS * (kpw0 + kpw1), extra, 0)

            def step(t, carry):
                run_batch(t * NBUF, NBUF)
                return carry

            lax.fori_loop(0, nfull, step, 0)
            if tail:
                run_batch(nfull * NBUF, tail)


@functools.partial(jax.jit, static_argnames=("n_pad", "rows"))
def _deg_pass(ei3, n_pad, rows):
    rpt = n_pad // NS
    kpw0, kpw1, extra = _partition(rows)
    kmax = max(kpw0, kpw1, extra)

    def body(ei_hbm, out_hbm, dst_v, ones_v, stage_v, acc_sh, sem):
        c = lax.axis_index("c")
        s = lax.axis_index("s")

        def zr(i, carry):
            stage_v[i] = jnp.zeros((F,), jnp.float32)
            return carry

        lax.fori_loop(0, rpt, zr, 0)

        def on(i, carry):
            ones_v[i] = jnp.ones((F,), jnp.float32)
            return carry

        lax.fori_loop(0, LANES, on, 0)
        pltpu.sync_copy(stage_v, acc_sh.at[pl.ds(s * rpt, rpt)])
        plsc.subcore_barrier()

        def idx_stage(base, count, off):
            pltpu.sync_copy(
                ei_hbm.at[1, pl.ds(base, count)], dst_v.at[pl.ds(off, count)]
            )

        def run_batch(j0, nb):
            descs = [
                pltpu.async_copy(
                    ones_v, acc_sh.at[dst_v.at[j0 + b]], sem, add=True
                )
                for b in range(nb)
            ]
            for desc in descs:
                desc.wait()

        _edge_loop(run_batch, idx_stage, ei_hbm, kpw0, kpw1, extra, c, s)

        plsc.subcore_barrier()
        pltpu.sync_copy(acc_sh.at[pl.ds(s * rpt, rpt)], stage_v)
        pltpu.sync_copy(stage_v, out_hbm.at[c, pl.ds(s * rpt, rpt)])

    return pl.kernel(
        body,
        out_type=jax.ShapeDtypeStruct((NC, n_pad, F), jnp.float32),
        mesh=_mesh(),
        compiler_params=pltpu.CompilerParams(use_tc_tiling_on_sc=False),
        scratch_types=[
            pltpu.VMEM((kmax, LANES), jnp.int32),
            pltpu.VMEM((LANES, F), jnp.float32),
            pltpu.VMEM((rpt, F), jnp.float32),
            pltpu.VMEM_SHARED((n_pad, F), jnp.float32),
            pltpu.SemaphoreType.DMA,
        ],
    )(ei3)


@functools.partial(jax.jit, static_argnames=("n_pad", "rows"))
def _agg_pass(hs, ei3, n_pad, rows):
    rpt = n_pad // NS
    kpw0, kpw1, extra = _partition(rows)
    kmax = max(kpw0, kpw1, extra)

    def body(hs_hbm, ei_hbm, out_hbm, src_v, dst_v, rows_v, stage_v,
             acc_sh, gsem, ssem):
        c = lax.axis_index("c")
        s = lax.axis_index("s")

        def zr(i, carry):
            stage_v[i] = jnp.zeros((F,), jnp.float32)
            return carry

        lax.fori_loop(0, rpt, zr, 0)
        pltpu.sync_copy(stage_v, acc_sh.at[pl.ds(s * rpt, rpt)])
        plsc.subcore_barrier()

        def idx_stage(base, count, off):
            pltpu.sync_copy(
                ei_hbm.at[0, pl.ds(base, count)], src_v.at[pl.ds(off, count)]
            )
            pltpu.sync_copy(
                ei_hbm.at[1, pl.ds(base, count)], dst_v.at[pl.ds(off, count)]
            )

        def run_batch(j0, nb):
            gds = [
                pltpu.async_copy(
                    hs_hbm.at[src_v.at[j0 + b]],
                    rows_v.at[pl.ds(b * LANES, LANES)],
                    gsem,
                )
                for b in range(nb)
            ]
            for desc in gds:
                desc.wait()
            sds = [
                pltpu.async_copy(
                    rows_v.at[pl.ds(b * LANES, LANES)],
                    acc_sh.at[dst_v.at[j0 + b]],
                    ssem,
                    add=True,
                )
                for b in range(nb)
            ]
            for desc in sds:
                desc.wait()

        _edge_loop(run_batch, idx_stage, ei_hbm, kpw0, kpw1, extra, c, s)

        plsc.subcore_barrier()
        pltpu.sync_copy(acc_sh.at[pl.ds(s * rpt, rpt)], stage_v)
        pltpu.sync_copy(stage_v, out_hbm.at[c, pl.ds(s * rpt, rpt)])

    return pl.kernel(
        body,
        out_type=jax.ShapeDtypeStruct((NC, n_pad, F), jnp.float32),
        mesh=_mesh(),
        compiler_params=pltpu.CompilerParams(use_tc_tiling_on_sc=False),
        scratch_types=[
            pltpu.VMEM((kmax, LANES), jnp.int32),
            pltpu.VMEM((kmax, LANES), jnp.int32),
            pltpu.VMEM((NBUF * LANES, F), jnp.float32),
            pltpu.VMEM((rpt, F), jnp.float32),
            pltpu.VMEM_SHARED((n_pad, F), jnp.float32),
            pltpu.SemaphoreType.DMA,
            pltpu.SemaphoreType.DMA,
        ],
    )(hs, ei3)


def _mm1_body(x_ref, w_ref, h_ref):
    h_ref[...] = jnp.dot(
        x_ref[...], w_ref[...], preferred_element_type=jnp.float32
    )


def _mm2_body(s_ref, w_ref, b_ref, g_ref, out_ref):
    logits = (
        jnp.dot(s_ref[...], w_ref[...], preferred_element_type=jnp.float32)
        + b_ref[...]
    )
    m = jnp.max(logits, axis=1, keepdims=True)
    e = jnp.exp(logits - m)
    sums = jnp.dot(e, g_ref[...], preferred_element_type=jnp.float32)
    out_ref[...] = (logits - m) - jnp.log(sums)


def _full_spec(shape):
    return pl.BlockSpec(shape, lambda i: tuple(0 for _ in shape))


def kernel(x, edge_index, W1, b1, W2, b2):
    n, d = x.shape
    c_dim = W2.shape[1]
    e = edge_index.shape[1]
    assert W1.shape[1] == F

    if e % LANES == 0:
        ei = edge_index
        n_pad = -(-n // LANES) * LANES
    else:
        pad = LANES - e % LANES
        n_pad = (n + LANES) // LANES * LANES
        ei = jnp.concatenate(
            [
                edge_index,
                jnp.tile(
                    jnp.array([[0], [n]], jnp.int32), (1, pad)
                ),
            ],
            axis=1,
        )
    rows = ei.shape[1] // LANES
    ei3 = ei.reshape(2, rows, LANES)

    pk = 128 // F
    assert n % pk == 0 and d % 8 == 0

    def packed(a2d):
        return a2d.reshape(a2d.shape[0] // pk, 128)

    eye = jnp.eye(pk, dtype=jnp.float32)
    w1big = jnp.kron(eye, W1)
    degp = _deg_pass(ei3, n_pad=n_pad, rows=rows)
    h1p = pl.pallas_call(
        _mm1_body,
        out_shape=jax.ShapeDtypeStruct((n // pk, 128), jnp.float32),
    )(x.reshape(n // pk, pk * d), w1big)

    b1t = jnp.tile(b1, pk)
    dis = lax.rsqrt(packed(degp[0, :n]) + packed(degp[1, :n]) + 1.0)
    hs1 = dis * h1p

    t1 = _agg_pass(hs1.reshape(n, F), ei3, n_pad=n_pad, rows=rows)

    r = jnp.maximum(
        dis * (packed(t1[0, :n]) + packed(t1[1, :n]) + hs1) + b1t, 0.0
    )
    hs2 = dis * r

    t2 = _agg_pass(hs2.reshape(n, F), ei3, n_pad=n_pad, rows=rows)

    s2 = dis * (packed(t2[0, :n]) + packed(t2[1, :n]) + hs2)

    w2big = jnp.kron(eye, W2)
    gmat = jnp.kron(eye, jnp.ones((c_dim, c_dim), jnp.float32))
    outp = pl.pallas_call(
        _mm2_body,
        out_shape=jax.ShapeDtypeStruct((n // pk, pk * c_dim), jnp.float32),
    )(s2, w2big, jnp.tile(b2, pk).reshape(1, pk * c_dim), gmat)

    return outp.reshape(n, c_dim)


# speedup vs baseline: 57.4808x; 1.0708x over previous
import functools

import jax
import jax.numpy as jnp
from jax import lax
from jax.experimental import pallas as pl
from jax.experimental.pallas import tpu as pltpu
from jax.experimental.pallas import tpu_sc as plsc

NC = 2
NS = 16
LANES = 128
F = 16
NBUF = 4
SPLIT0 = 0.54


def _mesh():
    return plsc.VectorSubcoreMesh(
        core_axis_name="c", subcore_axis_name="s", num_cores=NC, num_subcores=NS
    )


def _partition(rows):
    kpw0 = max(NBUF, int(round(rows * SPLIT0 / (NS * NBUF))) * NBUF)
    kpw0 = min(kpw0, (rows // (NS * NBUF)) * NBUF)
    rem = rows - NS * kpw0
    kpw1 = max(0, (rem // NS) // NBUF * NBUF)
    extra = rows - NS * kpw0 - NS * kpw1
    return kpw0, kpw1, extra


def _edge_loop(run_batch, idx_stage, ei3, kpw0, kpw1, extra, c, s):
    def run(base, kpw):
        idx_stage(base, kpw, 0)

        def step(t, carry):
            run_batch(t * NBUF, NBUF)
            return carry

        lax.fori_loop(0, kpw // NBUF, step, 0)

    @pl.when(c == 0)
    def _():
        run(s * kpw0, kpw0)

    @pl.when(c == 1)
    def _():
        if kpw1 > 0:
            run(NS * kpw0 + s * kpw1, kpw1)

    if extra > 0:
        nfull = extra // NBUF
        tail = extra % NBUF

        @pl.when((c == 1) & (s == NS - 1))
        def _():
            idx_stage(NS * (kpw0 + kpw1), extra, 0)

            def step(t, carry):
                run_batch(t * NBUF, NBUF)
                return carry

            lax.fori_loop(0, nfull, step, 0)
            if tail:
                run_batch(nfull * NBUF, tail)


@functools.partial(jax.jit, static_argnames=("n_pad", "rows"))
def _deg_pass(ei3, n_pad, rows):
    rpt = n_pad // NS
    kpw0, kpw1, extra = _partition(rows)
    kmax = max(kpw0, kpw1, extra)

    def body(ei_hbm, out_hbm, dst_v, ones_v, stage_v, acc_sh, sem):
        c = lax.axis_index("c")
        s = lax.axis_index("s")

        def zr(i, carry):
            stage_v[i] = jnp.zeros((F,), jnp.float32)
            return carry

        lax.fori_loop(0, rpt, zr, 0)

        def on(i, carry):
            ones_v[i] = jnp.ones((F,), jnp.float32)
            return carry

        lax.fori_loop(0, LANES, on, 0)
        pltpu.sync_copy(stage_v, acc_sh.at[pl.ds(s * rpt, rpt)])
        plsc.subcore_barrier()

        def idx_stage(base, count, off):
            pltpu.sync_copy(
                ei_hbm.at[1, pl.ds(base, count)], dst_v.at[pl.ds(off, count)]
            )

        def run_batch(j0, nb):
            descs = [
                pltpu.async_copy(
                    ones_v, acc_sh.at[dst_v.at[j0 + b]], sem, add=True
                )
                for b in range(nb)
            ]
            for desc in descs:
                desc.wait()

        _edge_loop(run_batch, idx_stage, ei_hbm, kpw0, kpw1, extra, c, s)

        plsc.subcore_barrier()
        pltpu.sync_copy(acc_sh.at[pl.ds(s * rpt, rpt)], stage_v)
        pltpu.sync_copy(stage_v, out_hbm.at[c, pl.ds(s * rpt, rpt)])

    return pl.kernel(
        body,
        out_type=jax.ShapeDtypeStruct((NC, n_pad, F), jnp.float32),
        mesh=_mesh(),
        compiler_params=pltpu.CompilerParams(use_tc_tiling_on_sc=False),
        scratch_types=[
            pltpu.VMEM((kmax, LANES), jnp.int32),
            pltpu.VMEM((LANES, F), jnp.float32),
            pltpu.VMEM((rpt, F), jnp.float32),
            pltpu.VMEM_SHARED((n_pad, F), jnp.float32),
            pltpu.SemaphoreType.DMA,
        ],
    )(ei3)


@functools.partial(jax.jit, static_argnames=("n_pad", "rows"))
def _agg_pass(hs, ei3, n_pad, rows):
    rpt = n_pad // NS
    kpw0, kpw1, extra = _partition(rows)
    kmax = max(kpw0, kpw1, extra)

    def body(hs_hbm, ei_hbm, out_hbm, src_v, dst_v, rows_v, stage_v,
             acc_sh, *sems):
        gsems = sems[:NBUF]
        ssems = sems[NBUF:]
        c = lax.axis_index("c")
        s = lax.axis_index("s")

        def zr(i, carry):
            stage_v[i] = jnp.zeros((F,), jnp.float32)
            return carry

        lax.fori_loop(0, rpt, zr, 0)
        pltpu.sync_copy(stage_v, acc_sh.at[pl.ds(s * rpt, rpt)])
        plsc.subcore_barrier()

        def idx_stage(base, count, off):
            pltpu.sync_copy(
                ei_hbm.at[0, pl.ds(base, count)], src_v.at[pl.ds(off, count)]
            )
            pltpu.sync_copy(
                ei_hbm.at[1, pl.ds(base, count)], dst_v.at[pl.ds(off, count)]
            )

        def run_batch(j0, nb):
            gds = [
                pltpu.async_copy(
                    hs_hbm.at[src_v.at[j0 + b]],
                    rows_v.at[pl.ds(b * LANES, LANES)],
                    gsems[b],
                )
                for b in range(nb)
            ]
            sds = []
            for b in range(nb):
                gds[b].wait()
                sds.append(
                    pltpu.async_copy(
                        rows_v.at[pl.ds(b * LANES, LANES)],
                        acc_sh.at[dst_v.at[j0 + b]],
                        ssems[b],
                        add=True,
                    )
                )
            for desc in sds:
                desc.wait()

        _edge_loop(run_batch, idx_stage, ei_hbm, kpw0, kpw1, extra, c, s)

        plsc.subcore_barrier()
        pltpu.sync_copy(acc_sh.at[pl.ds(s * rpt, rpt)], stage_v)
        pltpu.sync_copy(stage_v, out_hbm.at[c, pl.ds(s * rpt, rpt)])

    return pl.kernel(
        body,
        out_type=jax.ShapeDtypeStruct((NC, n_pad, F), jnp.float32),
        mesh=_mesh(),
        compiler_params=pltpu.CompilerParams(use_tc_tiling_on_sc=False),
        scratch_types=[
            pltpu.VMEM((kmax, LANES), jnp.int32),
            pltpu.VMEM((kmax, LANES), jnp.int32),
            pltpu.VMEM((NBUF * LANES, F), jnp.float32),
            pltpu.VMEM((rpt, F), jnp.float32),
            pltpu.VMEM_SHARED((n_pad, F), jnp.float32),
        ]
        + [pltpu.SemaphoreType.DMA] * (2 * NBUF),
    )(hs, ei3)


def _mm1_body(x_ref, w_ref, h_ref):
    h_ref[...] = jnp.dot(
        x_ref[...], w_ref[...], preferred_element_type=jnp.float32
    )


def _mm2_body(s_ref, w_ref, b_ref, g_ref, out_ref):
    logits = (
        jnp.dot(s_ref[...], w_ref[...], preferred_element_type=jnp.float32)
        + b_ref[...]
    )
    m = jnp.max(logits, axis=1, keepdims=True)
    e = jnp.exp(logits - m)
    sums = jnp.dot(e, g_ref[...], preferred_element_type=jnp.float32)
    out_ref[...] = (logits - m) - jnp.log(sums)


def _full_spec(shape):
    return pl.BlockSpec(shape, lambda i: tuple(0 for _ in shape))


def kernel(x, edge_index, W1, b1, W2, b2):
    n, d = x.shape
    c_dim = W2.shape[1]
    e = edge_index.shape[1]
    assert W1.shape[1] == F

    if e % LANES == 0:
        ei = edge_index
        n_pad = -(-n // LANES) * LANES
    else:
        pad = LANES - e % LANES
        n_pad = (n + LANES) // LANES * LANES
        ei = jnp.concatenate(
            [
                edge_index,
                jnp.tile(
                    jnp.array([[0], [n]], jnp.int32), (1, pad)
                ),
            ],
            axis=1,
        )
    rows = ei.shape[1] // LANES
    ei3 = ei.reshape(2, rows, LANES)

    pk = 128 // F
    assert n % pk == 0 and d % 8 == 0

    def packed(a2d):
        return a2d.reshape(a2d.shape[0] // pk, 128)[: n // pk]

    eye = jnp.eye(pk, dtype=jnp.float32)
    w1big = jnp.kron(eye, W1)
    degp = _deg_pass(ei3, n_pad=n_pad, rows=rows)
    h1p = pl.pallas_call(
        _mm1_body,
        out_shape=jax.ShapeDtypeStruct((n // pk, 128), jnp.float32),
    )(x.reshape(n // pk, pk * d), w1big)

    b1t = jnp.tile(b1, pk)
    dis = lax.rsqrt(packed(degp[0]) + packed(degp[1]) + 1.0)
    hs1 = dis * h1p

    t1 = _agg_pass(hs1.reshape(n, F), ei3, n_pad=n_pad, rows=rows)

    r = jnp.maximum(
        dis * (packed(t1[0]) + packed(t1[1]) + hs1) + b1t, 0.0
    )
    hs2 = dis * r

    t2 = _agg_pass(hs2.reshape(n, F), ei3, n_pad=n_pad, rows=rows)

    s2 = dis * (packed(t2[0]) + packed(t2[1]) + hs2)

    w2big = jnp.kron(eye, W2)
    gmat = jnp.kron(eye, jnp.ones((c_dim, c_dim), jnp.float32))
    outp = pl.pallas_call(
        _mm2_body,
        out_shape=jax.ShapeDtypeStruct((n // pk, pk * c_dim), jnp.float32),
    )(s2, w2big, jnp.tile(b2, pk).reshape(1, pk * c_dim), gmat)

    return outp.reshape(n, c_dim)


# speedup vs baseline: 76.3905x; 1.3290x over previous
import functools

import jax
import jax.numpy as jnp
from jax import lax
from jax.experimental import pallas as pl
from jax.experimental.pallas import tpu as pltpu
from jax.experimental.pallas import tpu_sc as plsc

NC = 2
NS = 16
LANES = 128
F = 16
NBUF = 4
SPLIT0 = 0.54


def _mesh():
    return plsc.VectorSubcoreMesh(
        core_axis_name="c", subcore_axis_name="s", num_cores=NC, num_subcores=NS
    )


def _partition(rows):
    kpw0 = max(NBUF, int(round(rows * SPLIT0 / (NS * NBUF))) * NBUF)
    kpw0 = min(kpw0, (rows // (NS * NBUF)) * NBUF)
    rem = rows - NS * kpw0
    kpw1 = max(0, (rem // NS) // NBUF * NBUF)
    extra = rows - NS * kpw0 - NS * kpw1
    return kpw0, kpw1, extra


def _emit_packed(acc_sh, stage_v, wide_v, out_hbm, rpt, c, s):
    pk = 128 // F
    wrows = rpt * F // 128
    pltpu.sync_copy(acc_sh.at[pl.ds(s * rpt, rpt)], stage_v)

    def rp(i, carry):
        for j in range(pk):
            wide_v[i, pl.ds(j * F, F)] = stage_v[i * pk + j]
        return carry

    lax.fori_loop(0, wrows, rp, 0)
    pltpu.sync_copy(wide_v, out_hbm.at[c, pl.ds(s * wrows, wrows)])


def _edge_loop(run_batch, idx_stage, ei3, kpw0, kpw1, extra, c, s):
    def run(base, kpw):
        idx_stage(base, kpw, 0)

        def step(t, carry):
            run_batch(t * NBUF, NBUF)
            return carry

        lax.fori_loop(0, kpw // NBUF, step, 0)

    @pl.when(c == 0)
    def _():
        run(s * kpw0, kpw0)

    @pl.when(c == 1)
    def _():
        if kpw1 > 0:
            run(NS * kpw0 + s * kpw1, kpw1)

    if extra > 0:
        nfull = extra // NBUF
        tail = extra % NBUF

        @pl.when((c == 1) & (s == NS - 1))
        def _():
            idx_stage(NS * (kpw0 + kpw1), extra, 0)

            def step(t, carry):
                run_batch(t * NBUF, NBUF)
                return carry

            lax.fori_loop(0, nfull, step, 0)
            if tail:
                run_batch(nfull * NBUF, tail)


@functools.partial(jax.jit, static_argnames=("n_pad", "rows"))
def _deg_pass(ei3, n_pad, rows):
    rpt = n_pad // NS
    kpw0, kpw1, extra = _partition(rows)
    kmax = max(kpw0, kpw1, extra)

    def body(ei_hbm, out_hbm, dst_v, ones_v, stage_v, wide_v, acc_sh, sem):
        c = lax.axis_index("c")
        s = lax.axis_index("s")

        def zr(i, carry):
            stage_v[i] = jnp.zeros((F,), jnp.float32)
            return carry

        lax.fori_loop(0, rpt, zr, 0)

        def on(i, carry):
            ones_v[i] = jnp.ones((F,), jnp.float32)
            return carry

        lax.fori_loop(0, LANES, on, 0)
        pltpu.sync_copy(stage_v, acc_sh.at[pl.ds(s * rpt, rpt)])
        plsc.subcore_barrier()

        def idx_stage(base, count, off):
            pltpu.sync_copy(
                ei_hbm.at[1, pl.ds(base, count)], dst_v.at[pl.ds(off, count)]
            )

        def run_batch(j0, nb):
            descs = [
                pltpu.async_copy(
                    ones_v, acc_sh.at[dst_v.at[j0 + b]], sem, add=True
                )
                for b in range(nb)
            ]
            for desc in descs:
                desc.wait()

        _edge_loop(run_batch, idx_stage, ei_hbm, kpw0, kpw1, extra, c, s)

        plsc.subcore_barrier()
        _emit_packed(acc_sh, stage_v, wide_v, out_hbm, rpt, c, s)

    return pl.kernel(
        body,
        out_type=jax.ShapeDtypeStruct((NC, n_pad * F // 128, 128), jnp.float32),
        mesh=_mesh(),
        compiler_params=pltpu.CompilerParams(use_tc_tiling_on_sc=False),
        scratch_types=[
            pltpu.VMEM((kmax, LANES), jnp.int32),
            pltpu.VMEM((LANES, F), jnp.float32),
            pltpu.VMEM((rpt, F), jnp.float32),
            pltpu.VMEM((rpt * F // 128, 128), jnp.float32),
            pltpu.VMEM_SHARED((n_pad, F), jnp.float32),
            pltpu.SemaphoreType.DMA,
        ],
    )(ei3)


@functools.partial(jax.jit, static_argnames=("n_pad", "rows"))
def _agg_pass(hs, ei3, n_pad, rows):
    rpt = n_pad // NS
    kpw0, kpw1, extra = _partition(rows)
    kmax = max(kpw0, kpw1, extra)

    def body(hs_hbm, ei_hbm, out_hbm, src_v, dst_v, rows_v, stage_v,
             wide_v, acc_sh, *sems):
        gsems = sems[:NBUF]
        ssems = sems[NBUF:]
        c = lax.axis_index("c")
        s = lax.axis_index("s")

        def zr(i, carry):
            stage_v[i] = jnp.zeros((F,), jnp.float32)
            return carry

        lax.fori_loop(0, rpt, zr, 0)
        pltpu.sync_copy(stage_v, acc_sh.at[pl.ds(s * rpt, rpt)])
        plsc.subcore_barrier()

        def idx_stage(base, count, off):
            pltpu.sync_copy(
                ei_hbm.at[0, pl.ds(base, count)], src_v.at[pl.ds(off, count)]
            )
            pltpu.sync_copy(
                ei_hbm.at[1, pl.ds(base, count)], dst_v.at[pl.ds(off, count)]
            )

        def run_batch(j0, nb):
            gds = [
                pltpu.async_copy(
                    hs_hbm.at[src_v.at[j0 + b]],
                    rows_v.at[pl.ds(b * LANES, LANES)],
                    gsems[b],
                )
                for b in range(nb)
            ]
            sds = []
            for b in range(nb):
                gds[b].wait()
                sds.append(
                    pltpu.async_copy(
                        rows_v.at[pl.ds(b * LANES, LANES)],
                        acc_sh.at[dst_v.at[j0 + b]],
                        ssems[b],
                        add=True,
                    )
                )
            for desc in sds:
                desc.wait()

        _edge_loop(run_batch, idx_stage, ei_hbm, kpw0, kpw1, extra, c, s)

        plsc.subcore_barrier()
        _emit_packed(acc_sh, stage_v, wide_v, out_hbm, rpt, c, s)

    return pl.kernel(
        body,
        out_type=jax.ShapeDtypeStruct((NC, n_pad * F // 128, 128), jnp.float32),
        mesh=_mesh(),
        compiler_params=pltpu.CompilerParams(use_tc_tiling_on_sc=False),
        scratch_types=[
            pltpu.VMEM((kmax, LANES), jnp.int32),
            pltpu.VMEM((kmax, LANES), jnp.int32),
            pltpu.VMEM((NBUF * LANES, F), jnp.float32),
            pltpu.VMEM((rpt, F), jnp.float32),
            pltpu.VMEM((rpt * F // 128, 128), jnp.float32),
            pltpu.VMEM_SHARED((n_pad, F), jnp.float32),
        ]
        + [pltpu.SemaphoreType.DMA] * (2 * NBUF),
    )(hs, ei3)


def _mm1_body(x_ref, w_ref, h_ref):
    h_ref[...] = jnp.dot(
        x_ref[...], w_ref[...], preferred_element_type=jnp.float32
    )


def _mm2_body(s_ref, w_ref, b_ref, g_ref, out_ref):
    logits = (
        jnp.dot(s_ref[...], w_ref[...], preferred_element_type=jnp.float32)
        + b_ref[...]
    )
    m = jnp.max(logits, axis=1, keepdims=True)
    e = jnp.exp(logits - m)
    sums = jnp.dot(e, g_ref[...], preferred_element_type=jnp.float32)
    out_ref[...] = (logits - m) - jnp.log(sums)


def _full_spec(shape):
    return pl.BlockSpec(shape, lambda i: tuple(0 for _ in shape))


def kernel(x, edge_index, W1, b1, W2, b2):
    n, d = x.shape
    c_dim = W2.shape[1]
    e = edge_index.shape[1]
    assert W1.shape[1] == F

    if e % LANES == 0:
        ei = edge_index
        n_pad = -(-n // LANES) * LANES
    else:
        pad = LANES - e % LANES
        n_pad = (n + LANES) // LANES * LANES
        ei = jnp.concatenate(
            [
                edge_index,
                jnp.tile(
                    jnp.array([[0], [n]], jnp.int32), (1, pad)
                ),
            ],
            axis=1,
        )
    rows = ei.shape[1] // LANES
    ei3 = ei.reshape(2, rows, LANES)

    pk = 128 // F
    assert n % pk == 0 and d % 8 == 0

    def packed(a2d):
        return a2d[: n // pk]

    eye = jnp.eye(pk, dtype=jnp.float32)
    w1big = jnp.kron(eye, W1)
    degp = _deg_pass(ei3, n_pad=n_pad, rows=rows)
    h1p = pl.pallas_call(
        _mm1_body,
        out_shape=jax.ShapeDtypeStruct((n // pk, 128), jnp.float32),
    )(x.reshape(n // pk, pk * d), w1big)

    b1t = jnp.tile(b1, pk)
    dis = lax.rsqrt(packed(degp[0]) + packed(degp[1]) + 1.0)
    hs1 = dis * h1p

    t1 = _agg_pass(hs1.reshape(n, F), ei3, n_pad=n_pad, rows=rows)

    r = jnp.maximum(
        dis * (packed(t1[0]) + packed(t1[1]) + hs1) + b1t, 0.0
    )
    hs2 = dis * r

    t2 = _agg_pass(hs2.reshape(n, F), ei3, n_pad=n_pad, rows=rows)

    s2 = dis * (packed(t2[0]) + packed(t2[1]) + hs2)

    w2big = jnp.kron(eye, W2)
    gmat = jnp.kron(eye, jnp.ones((c_dim, c_dim), jnp.float32))
    outp = pl.pallas_call(
        _mm2_body,
        out_shape=jax.ShapeDtypeStruct((n // pk, pk * c_dim), jnp.float32),
    )(s2, w2big, jnp.tile(b2, pk).reshape(1, pk * c_dim), gmat)

    return outp.reshape(n, c_dim)


# speedup vs baseline: 78.2439x; 1.0243x over previous
import functools

import jax
import jax.numpy as jnp
from jax import lax
from jax.experimental import pallas as pl
from jax.experimental.pallas import tpu as pltpu
from jax.experimental.pallas import tpu_sc as plsc

NC = 2
NS = 16
LANES = 128
F = 16
NBUF = 4
SPLIT0 = 0.515


def _mesh():
    return plsc.VectorSubcoreMesh(
        core_axis_name="c", subcore_axis_name="s", num_cores=NC, num_subcores=NS
    )


def _partition(rows):
    kpw0 = max(NBUF, int(round(rows * SPLIT0 / (NS * NBUF))) * NBUF)
    kpw0 = min(kpw0, (rows // (NS * NBUF)) * NBUF)
    rem = rows - NS * kpw0
    kpw1 = max(0, (rem // NS) // NBUF * NBUF)
    extra = rows - NS * kpw0 - NS * kpw1
    return kpw0, kpw1, extra


def _emit_packed(acc_sh, stage_v, wide_v, out_hbm, rpt, c, s):
    pk = 128 // F
    wrows = rpt * F // 128
    pltpu.sync_copy(acc_sh.at[pl.ds(s * rpt, rpt)], stage_v)

    def rp(i, carry):
        for j in range(pk):
            wide_v[i, pl.ds(j * F, F)] = stage_v[i * pk + j]
        return carry

    lax.fori_loop(0, wrows, rp, 0)
    pltpu.sync_copy(wide_v, out_hbm.at[c, pl.ds(s * wrows, wrows)])


def _edge_loop(run_batch, idx_stage, ei3, kpw0, kpw1, extra, c, s):
    def run(base, kpw):
        idx_stage(base, kpw, 0)

        def step(t, carry):
            run_batch(t * NBUF, NBUF)
            return carry

        lax.fori_loop(0, kpw // NBUF, step, 0)

    @pl.when(c == 0)
    def _():
        run(s * kpw0, kpw0)

    @pl.when(c == 1)
    def _():
        if kpw1 > 0:
            run(NS * kpw0 + s * kpw1, kpw1)

    if extra > 0:
        nfull = extra // NBUF
        tail = extra % NBUF

        @pl.when((c == 1) & (s == NS - 1))
        def _():
            idx_stage(NS * (kpw0 + kpw1), extra, 0)

            def step(t, carry):
                run_batch(t * NBUF, NBUF)
                return carry

            lax.fori_loop(0, nfull, step, 0)
            if tail:
                run_batch(nfull * NBUF, tail)


@functools.partial(jax.jit, static_argnames=("n_pad", "rows"))
def _deg_pass(ei3, n_pad, rows):
    rpt = n_pad // NS
    kpw0, kpw1, extra = _partition(rows)
    kmax = max(kpw0, kpw1, extra)

    def body(ei_hbm, out_hbm, dst_v, ones_v, stage_v, wide_v, acc_sh, sem):
        c = lax.axis_index("c")
        s = lax.axis_index("s")

        def zr(i, carry):
            stage_v[i] = jnp.zeros((F,), jnp.float32)
            return carry

        lax.fori_loop(0, rpt, zr, 0)

        def on(i, carry):
            ones_v[i] = jnp.ones((F,), jnp.float32)
            return carry

        lax.fori_loop(0, LANES, on, 0)
        pltpu.sync_copy(stage_v, acc_sh.at[pl.ds(s * rpt, rpt)])
        plsc.subcore_barrier()

        def idx_stage(base, count, off):
            pltpu.sync_copy(
                ei_hbm.at[1, pl.ds(base, count)], dst_v.at[pl.ds(off, count)]
            )

        def run_batch(j0, nb):
            descs = [
                pltpu.async_copy(
                    ones_v, acc_sh.at[dst_v.at[j0 + b]], sem, add=True
                )
                for b in range(nb)
            ]
            for desc in descs:
                desc.wait()

        _edge_loop(run_batch, idx_stage, ei_hbm, kpw0, kpw1, extra, c, s)

        plsc.subcore_barrier()
        _emit_packed(acc_sh, stage_v, wide_v, out_hbm, rpt, c, s)

    return pl.kernel(
        body,
        out_type=jax.ShapeDtypeStruct((NC, n_pad * F // 128, 128), jnp.float32),
        mesh=_mesh(),
        compiler_params=pltpu.CompilerParams(use_tc_tiling_on_sc=False),
        scratch_types=[
            pltpu.VMEM((kmax, LANES), jnp.int32),
            pltpu.VMEM((LANES, F), jnp.float32),
            pltpu.VMEM((rpt, F), jnp.float32),
            pltpu.VMEM((rpt * F // 128, 128), jnp.float32),
            pltpu.VMEM_SHARED((n_pad, F), jnp.float32),
            pltpu.SemaphoreType.DMA,
        ],
    )(ei3)


@functools.partial(jax.jit, static_argnames=("n_pad", "rows"))
def _agg_pass(hs, ei3, n_pad, rows):
    rpt = n_pad // NS
    kpw0, kpw1, extra = _partition(rows)
    kmax = max(kpw0, kpw1, extra)

    def body(hs_hbm, ei_hbm, out_hbm, src_v, dst_v, rows_v, stage_v,
             wide_v, acc_sh, *sems):
        gsems = sems[:NBUF]
        ssems = sems[NBUF:]
        c = lax.axis_index("c")
        s = lax.axis_index("s")

        def zr(i, carry):
            stage_v[i] = jnp.zeros((F,), jnp.float32)
            return carry

        lax.fori_loop(0, rpt, zr, 0)
        pltpu.sync_copy(stage_v, acc_sh.at[pl.ds(s * rpt, rpt)])
        plsc.subcore_barrier()

        def idx_stage(base, count, off):
            pltpu.sync_copy(
                ei_hbm.at[0, pl.ds(base, count)], src_v.at[pl.ds(off, count)]
            )
            pltpu.sync_copy(
                ei_hbm.at[1, pl.ds(base, count)], dst_v.at[pl.ds(off, count)]
            )

        def run_batch(j0, nb):
            gds = [
                pltpu.async_copy(
                    hs_hbm.at[src_v.at[j0 + b]],
                    rows_v.at[pl.ds(b * LANES, LANES)],
                    gsems[b],
                )
                for b in range(nb)
            ]
            sds = []
            for b in range(nb):
                gds[b].wait()
                sds.append(
                    pltpu.async_copy(
                        rows_v.at[pl.ds(b * LANES, LANES)],
                        acc_sh.at[dst_v.at[j0 + b]],
                        ssems[b],
                        add=True,
                    )
                )
            for desc in sds:
                desc.wait()

        _edge_loop(run_batch, idx_stage, ei_hbm, kpw0, kpw1, extra, c, s)

        plsc.subcore_barrier()
        _emit_packed(acc_sh, stage_v, wide_v, out_hbm, rpt, c, s)

    return pl.kernel(
        body,
        out_type=jax.ShapeDtypeStruct((NC, n_pad * F // 128, 128), jnp.float32),
        mesh=_mesh(),
        compiler_params=pltpu.CompilerParams(use_tc_tiling_on_sc=False),
        scratch_types=[
            pltpu.VMEM((kmax, LANES), jnp.int32),
            pltpu.VMEM((kmax, LANES), jnp.int32),
            pltpu.VMEM((NBUF * LANES, F), jnp.float32),
            pltpu.VMEM((rpt, F), jnp.float32),
            pltpu.VMEM((rpt * F // 128, 128), jnp.float32),
            pltpu.VMEM_SHARED((n_pad, F), jnp.float32),
        ]
        + [pltpu.SemaphoreType.DMA] * (2 * NBUF),
    )(hs, ei3)


def _mm1_body(x_ref, w_ref, h_ref):
    h_ref[...] = jnp.dot(
        x_ref[...], w_ref[...], preferred_element_type=jnp.float32
    )


def _mm2_body(s_ref, w_ref, b_ref, g_ref, out_ref):
    logits = (
        jnp.dot(s_ref[...], w_ref[...], preferred_element_type=jnp.float32)
        + b_ref[...]
    )
    m = jnp.max(logits, axis=1, keepdims=True)
    e = jnp.exp(logits - m)
    sums = jnp.dot(e, g_ref[...], preferred_element_type=jnp.float32)
    out_ref[...] = (logits - m) - jnp.log(sums)


def _full_spec(shape):
    return pl.BlockSpec(shape, lambda i: tuple(0 for _ in shape))


def kernel(x, edge_index, W1, b1, W2, b2):
    n, d = x.shape
    c_dim = W2.shape[1]
    e = edge_index.shape[1]
    assert W1.shape[1] == F

    if e % LANES == 0:
        ei = edge_index
        n_pad = -(-n // LANES) * LANES
    else:
        pad = LANES - e % LANES
        n_pad = (n + LANES) // LANES * LANES
        ei = jnp.concatenate(
            [
                edge_index,
                jnp.tile(
                    jnp.array([[0], [n]], jnp.int32), (1, pad)
                ),
            ],
            axis=1,
        )
    rows = ei.shape[1] // LANES
    ei3 = ei.reshape(2, rows, LANES)

    pk = 128 // F
    assert n % pk == 0 and d % 8 == 0

    def packed(a2d):
        return a2d[: n // pk]

    eye = jnp.eye(pk, dtype=jnp.float32)
    w1big = jnp.kron(eye, W1)
    degp = _deg_pass(ei3, n_pad=n_pad, rows=rows)
    h1p = pl.pallas_call(
        _mm1_body,
        out_shape=jax.ShapeDtypeStruct((n // pk, 128), jnp.float32),
    )(x.reshape(n // pk, pk * d), w1big)

    b1t = jnp.tile(b1, pk)
    dis = lax.rsqrt(packed(degp[0]) + packed(degp[1]) + 1.0)
    hs1 = dis * h1p

    t1 = _agg_pass(hs1.reshape(n, F), ei3, n_pad=n_pad, rows=rows)

    r = jnp.maximum(
        dis * (packed(t1[0]) + packed(t1[1]) + hs1) + b1t, 0.0
    )
    hs2 = dis * r

    t2 = _agg_pass(hs2.reshape(n, F), ei3, n_pad=n_pad, rows=rows)

    s2 = dis * (packed(t2[0]) + packed(t2[1]) + hs2)

    w2big = jnp.kron(eye, W2)
    gmat = jnp.kron(eye, jnp.ones((c_dim, c_dim), jnp.float32))
    outp = pl.pallas_call(
        _mm2_body,
        out_shape=jax.ShapeDtypeStruct((n // pk, pk * c_dim), jnp.float32),
    )(s2, w2big, jnp.tile(b2, pk).reshape(1, pk * c_dim), gmat)

    return outp.reshape(n, c_dim)
